# baseline (speedup 1.0000x reference)
import functools

import jax
import jax.numpy as jnp
from jax import lax
from jax.experimental import pallas as pl
from jax.experimental.pallas import tpu as pltpu
from jax.experimental.pallas import tpu_sc as plsc

N_TOKENS = 16384
EMBED = 2048
N_EXPERTS = 64
TOP_K = 8
BLK = 2048
NBLK = N_TOKENS // BLK

NW = 32
TPW = N_TOKENS // NW
NGRP = TPW // 16


def _logits_kernel(x_ref, w_ref, b_ref, lg_ref):
    lt = jax.lax.dot_general(
        w_ref[...], x_ref[...], (((1,), (1,)), ((), ())),
        preferred_element_type=jnp.float32,
        precision=jax.lax.Precision.DEFAULT,
    ) + b_ref[...]
    lg_ref[...] = lt.T


def _tc_logits(inputs, W, b2):
    return pl.pallas_call(
        _logits_kernel,
        grid=(NBLK,),
        in_specs=[
            pl.BlockSpec((BLK, EMBED), lambda i: (i, 0)),
            pl.BlockSpec((N_EXPERTS, EMBED), lambda i: (0, 0)),
            pl.BlockSpec((N_EXPERTS, 1), lambda i: (0, 0)),
        ],
        out_specs=pl.BlockSpec((BLK, N_EXPERTS), lambda i: (i, 0)),
        out_shape=jax.ShapeDtypeStruct((N_TOKENS, N_EXPERTS), jnp.float32),
    )(inputs, W, b2)


def _sc_topk_kernel(lg_hbm, probs_hbm, idx_hbm, lg_v, probs_v, idx_v):
    wid = lax.axis_index("c") * 16 + lax.axis_index("s")
    base = wid * TPW
    pltpu.sync_copy(lg_hbm.at[pl.ds(base * N_EXPERTS, TPW * N_EXPERTS)], lg_v)

    lanes = lax.iota(jnp.int32, 16)
    zeros = jnp.zeros((16,), jnp.float32)

    def zero_body(k, _):
        probs_v[pl.ds(k * 16, 16)] = zeros
        return 0

    lax.fori_loop(0, TPW * N_EXPERTS // 16, zero_body, 0)

    def group_body(g, _):
        tok = g * 16 + lanes
        neg = jnp.full((16,), -jnp.inf, jnp.float32)
        val = [neg] * TOP_K
        idx = [jnp.zeros((16,), jnp.int32)] * TOP_K
        tok64 = tok * N_EXPERTS
        for e in range(N_EXPERTS):
            v = plsc.load_gather(lg_v, [tok64 + e])
            ei = jnp.full((16,), e, jnp.int32)
            for r in range(TOP_K):
                gt = v > val[r]
                nv = jnp.where(gt, v, val[r])
                ni = jnp.where(gt, ei, idx[r])
                v, ei = jnp.where(gt, val[r], v), jnp.where(gt, idx[r], ei)
                val[r], idx[r] = nv, ni
        es = [jnp.exp(val[r] - val[0]) for r in range(TOP_K)]
        tot = es[0]
        for r in range(1, TOP_K):
            tot = tot + es[r]
        inv = 1.0 / tot
        tok8 = tok * TOP_K
        for r in range(TOP_K):
            plsc.store_scatter(probs_v, [tok64 + idx[r]], es[r] * inv)
            plsc.store_scatter(idx_v, [tok8 + r], idx[r])
        return 0

    lax.fori_loop(0, NGRP, group_body, 0)

    pltpu.sync_copy(probs_v, probs_hbm.at[pl.ds(base * N_EXPERTS, TPW * N_EXPERTS)])
    pltpu.sync_copy(idx_v, idx_hbm.at[pl.ds(base * TOP_K, TPW * TOP_K)])


_sc_topk = functools.partial(
    pl.kernel,
    mesh=plsc.VectorSubcoreMesh(core_axis_name="c", subcore_axis_name="s"),
    out_type=[
        jax.ShapeDtypeStruct((N_TOKENS * N_EXPERTS,), jnp.float32),
        jax.ShapeDtypeStruct((N_TOKENS * TOP_K,), jnp.int32),
    ],
    scratch_types=[
        pltpu.VMEM((TPW * N_EXPERTS,), jnp.float32),
        pltpu.VMEM((TPW * N_EXPERTS,), jnp.float32),
        pltpu.VMEM((TPW * TOP_K,), jnp.int32),
    ],
    compiler_params=pltpu.CompilerParams(
        use_tc_tiling_on_sc=False, needs_layout_passes=False),
)(_sc_topk_kernel)


@jax.jit
def kernel(inputs, W, b):
    b2 = b.reshape(N_EXPERTS, 1)
    logits = _tc_logits(inputs, W, b2)
    probs_flat, idx_flat = _sc_topk(logits.reshape(-1))
    return (probs_flat.reshape(N_TOKENS, N_EXPERTS),
            idx_flat.reshape(N_TOKENS, TOP_K))


# speedup vs baseline: 2.2340x; 2.2340x over previous
import jax
import jax.numpy as jnp
from jax.experimental import pallas as pl

N_TOKENS = 16384
EMBED = 2048
N_EXPERTS = 64
TOP_K = 8
BLK = 2048


def _router_kernel(x_ref, w_ref, b_ref, probs_ref, idx_ref):
    lt = jax.lax.dot_general(
        w_ref[...], x_ref[...], (((1,), (1,)), ((), ())),
        preferred_element_type=jnp.float32,
        precision=jax.lax.Precision.DEFAULT,
    ) + b_ref[...]

    iota0 = jax.lax.broadcasted_iota(jnp.int32, lt.shape, 0)
    neg = jnp.float32(-jnp.inf)
    cur = lt
    idx_rows = []
    for _ in range(TOP_K):
        m = jnp.max(cur, axis=0, keepdims=True)
        idx = jnp.min(jnp.where(cur == m, iota0, N_EXPERTS), axis=0, keepdims=True)
        cur = jnp.where(iota0 == idx, neg, cur)
        idx_rows.append(idx)
    idx_ref[...] = jnp.concatenate(idx_rows, axis=0).T

    selected = cur == neg
    mx = jnp.max(jnp.where(selected, lt, neg), axis=0, keepdims=True)
    e = jnp.where(selected, jnp.exp(lt - mx), 0.0)
    probs_ref[...] = (e / jnp.sum(e, axis=0, keepdims=True)).T


@jax.jit
def kernel(inputs, W, b):
    b2 = b.reshape(N_EXPERTS, 1)
    probs, idx = pl.pallas_call(
        _router_kernel,
        grid=(N_TOKENS // BLK,),
        in_specs=[
            pl.BlockSpec((BLK, EMBED), lambda i: (i, 0)),
            pl.BlockSpec((N_EXPERTS, EMBED), lambda i: (0, 0)),
            pl.BlockSpec((N_EXPERTS, 1), lambda i: (0, 0)),
        ],
        out_specs=[
            pl.BlockSpec((BLK, N_EXPERTS), lambda i: (i, 0)),
            pl.BlockSpec((BLK, TOP_K), lambda i: (i, 0)),
        ],
        out_shape=[
            jax.ShapeDtypeStruct((N_TOKENS, N_EXPERTS), jnp.float32),
            jax.ShapeDtypeStruct((N_TOKENS, TOP_K), jnp.int32),
        ],
    )(inputs, W, b2)
    return (probs, idx)
